# Initial kernel scaffold; baseline (speedup 1.0000x reference)
import functools

import jax
import jax.numpy as jnp
from jax import lax
from jax.experimental import pallas as pl
from jax.experimental.pallas import tpu as pltpu
from jax.experimental.pallas import tpu_sc as plsc

N = 10000
E = 320000
F_IN = 128
HID = 128
C = 40

NC = 2
NS = 16
NT = NC * NS
B = 125
EPT = E // NT
NB = EPT // B
ROWS_PT = N // NS

_MESH = plsc.VectorSubcoreMesh(core_axis_name="c", subcore_axis_name="s")


@functools.partial(
    pl.kernel,
    out_type=jax.ShapeDtypeStruct((NC, N), jnp.float32),
    mesh=_MESH,
    scratch_types=[
        pltpu.VMEM((NB, B), jnp.int32),
        pltpu.VMEM((B,), jnp.float32),
        pltpu.VMEM_SHARED((N,), jnp.float32),
    ],
)
def _deg_kernel(dst2d_hbm, ones_hbm, zeros_hbm, out_hbm, idx_v, ones_v, acc):
    cid = lax.axis_index("c")
    sid = lax.axis_index("s")
    tid = cid * NS + sid
    pltpu.sync_copy(dst2d_hbm.at[pl.ds(tid * NB, NB)], idx_v)
    pltpu.sync_copy(ones_hbm, ones_v)
    @pl.when(sid < NS - 1)
    def _():
        pltpu.sync_copy(zeros_hbm.at[pl.ds(sid * 640, 640)],
                        acc.at[pl.ds(sid * 640, 640)])

    @pl.when(sid == NS - 1)
    def _():
        pltpu.sync_copy(zeros_hbm.at[pl.ds(9600, 400)], acc.at[pl.ds(9600, 400)])

    plsc.subcore_barrier()

    def body(j, carry):
        pltpu.sync_copy(ones_v, acc.at[idx_v.at[j]], add=True)
        return carry

    lax.fori_loop(0, NB, body, 0)
    plsc.subcore_barrier()

    @pl.when(sid < NS - 1)
    def _():
        pltpu.sync_copy(acc.at[pl.ds(sid * 640, 640)],
                        out_hbm.at[cid, pl.ds(sid * 640, 640)])

    @pl.when(sid == NS - 1)
    def _():
        pltpu.sync_copy(acc.at[pl.ds(9600, 400)], out_hbm.at[cid, pl.ds(9600, 400)])


def _make_scatter(D):
    @functools.partial(
        pl.kernel,
        out_type=jax.ShapeDtypeStruct((NC, N, D), jnp.float32),
        mesh=_MESH,
        scratch_types=[
            pltpu.VMEM((NB, B), jnp.int32),
            pltpu.VMEM((NB, B), jnp.int32),
            pltpu.VMEM((B, D), jnp.float32),
            pltpu.VMEM((B, D), jnp.float32),
            pltpu.VMEM_SHARED((N, D), jnp.float32),
            pltpu.SemaphoreType.DMA,
            pltpu.SemaphoreType.DMA,
        ],
    )
    def _scatter(g_hbm, src2d_hbm, dst2d_hbm, zeros_hbm, out_hbm,
                 srcv, dstv, buf0, buf1, acc, sem0, sem1):
        cid = lax.axis_index("c")
        sid = lax.axis_index("s")
        tid = cid * NS + sid
        pltpu.sync_copy(src2d_hbm.at[pl.ds(tid * NB, NB)], srcv)
        pltpu.sync_copy(dst2d_hbm.at[pl.ds(tid * NB, NB)], dstv)
        pltpu.sync_copy(zeros_hbm.at[pl.ds(sid * ROWS_PT, ROWS_PT)],
                        acc.at[pl.ds(sid * ROWS_PT, ROWS_PT)])
        plsc.subcore_barrier()

        bufs = (buf0, buf1)
        sems = (sem0, sem1)
        pltpu.async_copy(g_hbm.at[srcv.at[0]], buf0, sem0)

        def body(j, carry):
            @pl.when(j % 2 == 0)
            def _():
                @pl.when(j + 1 < NB)
                def _():
                    pltpu.async_copy(g_hbm.at[srcv.at[j + 1]], buf1, sem1)

                pltpu.make_async_copy(g_hbm.at[srcv.at[j]], buf0, sem0).wait()
                pltpu.sync_copy(buf0, acc.at[dstv.at[j]], add=True)

            @pl.when(j % 2 == 1)
            def _():
                @pl.when(j + 1 < NB)
                def _():
                    pltpu.async_copy(g_hbm.at[srcv.at[j + 1]], buf0, sem0)

                pltpu.make_async_copy(g_hbm.at[srcv.at[j]], buf1, sem1).wait()
                pltpu.sync_copy(buf1, acc.at[dstv.at[j]], add=True)

            return carry

        lax.fori_loop(0, NB, body, 0)
        plsc.subcore_barrier()
        pltpu.sync_copy(acc.at[pl.ds(sid * ROWS_PT, ROWS_PT)],
                        out_hbm.at[cid, pl.ds(sid * ROWS_PT, ROWS_PT)])

    return _scatter


_scatter_hid = _make_scatter(HID)
_scatter_out = _make_scatter(C)


_R = 2000


def _tc1_body(x_ref, w_ref, degt_ref, g1_ref, dinv_ref):
    d2 = degt_ref[...]
    deg = d2[:, 0:1] + d2[:, 1:2] + 1.0
    dinv = lax.rsqrt(deg)
    h = jnp.dot(x_ref[...], w_ref[...], preferred_element_type=jnp.float32)
    g1_ref[...] = h * dinv
    dinv_ref[...] = dinv


def _tc1(x, w1, degt):
    return pl.pallas_call(
        _tc1_body,
        grid=(N // _R,),
        in_specs=[
            pl.BlockSpec((_R, F_IN), lambda i: (i, 0)),
            pl.BlockSpec((F_IN, HID), lambda i: (0, 0)),
            pl.BlockSpec((_R, NC), lambda i: (i, 0)),
        ],
        out_specs=[
            pl.BlockSpec((_R, HID), lambda i: (i, 0)),
            pl.BlockSpec((_R, 1), lambda i: (i, 0)),
        ],
        out_shape=[
            jax.ShapeDtypeStruct((N, HID), jnp.float32),
            jax.ShapeDtypeStruct((N, 1), jnp.float32),
        ],
    )(x, w1, degt)


def _tc2_body(acc_ref, g1_ref, dinv_ref, b1_ref, w2_ref, g2_ref):
    a = acc_ref[0] + acc_ref[1] + g1_ref[...]
    y = a * dinv_ref[...] + b1_ref[...]
    h = jnp.maximum(y, 0.0)
    g2_ref[...] = jnp.dot(h, w2_ref[...],
                          preferred_element_type=jnp.float32) * dinv_ref[...]


def _tc2(acc1, g1, dinv, b1, w2):
    return pl.pallas_call(
        _tc2_body,
        grid=(N // _R,),
        in_specs=[
            pl.BlockSpec((NC, _R, HID), lambda i: (0, i, 0)),
            pl.BlockSpec((_R, HID), lambda i: (i, 0)),
            pl.BlockSpec((_R, 1), lambda i: (i, 0)),
            pl.BlockSpec((1, HID), lambda i: (0, 0)),
            pl.BlockSpec((HID, C), lambda i: (0, 0)),
        ],
        out_specs=pl.BlockSpec((_R, C), lambda i: (i, 0)),
        out_shape=jax.ShapeDtypeStruct((N, C), jnp.float32),
    )(acc1, g1, dinv, b1, w2)


def _tc3_body(acc_ref, g2_ref, dinv_ref, b2_ref, o_ref):
    a = acc_ref[0] + acc_ref[1] + g2_ref[...]
    y = a * dinv_ref[...] + b2_ref[...]
    m = jnp.max(y, axis=1, keepdims=True)
    e = jnp.exp(y - m)
    s = jnp.sum(e, axis=1, keepdims=True)
    o_ref[...] = y - m - jnp.log(s)


def _tc3(acc2, g2, dinv, b2):
    return pl.pallas_call(
        _tc3_body,
        grid=(N // _R,),
        in_specs=[
            pl.BlockSpec((NC, _R, C), lambda i: (0, i, 0)),
            pl.BlockSpec((_R, C), lambda i: (i, 0)),
            pl.BlockSpec((_R, 1), lambda i: (i, 0)),
            pl.BlockSpec((1, C), lambda i: (0, 0)),
        ],
        out_specs=pl.BlockSpec((_R, C), lambda i: (i, 0)),
        out_shape=jax.ShapeDtypeStruct((N, C), jnp.float32),
    )(acc2, g2, dinv, b2)


def kernel(x, edge_index, W1, b1, W2, b2):
    src2d = edge_index[0].reshape(E // B, B)
    dst2d = edge_index[1].reshape(E // B, B)
    ones_b = jnp.ones((B,), jnp.float32)
    zeros_n = jnp.zeros((N,), jnp.float32)
    zeros_h = jnp.zeros((N, HID), jnp.float32)
    zeros_c = jnp.zeros((N, C), jnp.float32)

    degp = _deg_kernel(dst2d, ones_b, zeros_n)
    g1, dinv = _tc1(x, W1, degp.T)
    acc1 = _scatter_hid(g1, src2d, dst2d, zeros_h)
    g2 = _tc2(acc1, g1, dinv, b1.reshape(1, HID), W2)
    acc2 = _scatter_out(g2, src2d, dst2d, zeros_c)
    return _tc3(acc2, g2, dinv, b2.reshape(1, C))


# speedup vs baseline: 30.5860x; 30.5860x over previous
import functools

import jax
import jax.numpy as jnp
from jax import lax
from jax.experimental import pallas as pl
from jax.experimental.pallas import tpu as pltpu
from jax.experimental.pallas import tpu_sc as plsc

N = 10000
E = 320000
F_IN = 128
HID = 128
C = 40

NC = 2
NS = 16
NT = NC * NS
B = 125
EPT = E // NT
NB = EPT // B
NPAD = 10240
ROWS_PT = NPAD // NS

_MESH = plsc.VectorSubcoreMesh(core_axis_name="c", subcore_axis_name="s")


@functools.partial(
    pl.kernel,
    out_type=jax.ShapeDtypeStruct((NC, NPAD), jnp.float32),
    mesh=_MESH,
    scratch_types=[
        pltpu.VMEM((NB, B), jnp.int32),
        pltpu.VMEM((B,), jnp.float32),
        pltpu.VMEM_SHARED((NPAD,), jnp.float32),
    ],
)
def _deg_kernel(dst2d_hbm, ones_hbm, zeros_hbm, out_hbm, idx_v, ones_v, acc):
    cid = lax.axis_index("c")
    sid = lax.axis_index("s")
    tid = cid * NS + sid
    pltpu.sync_copy(dst2d_hbm.at[pl.ds(tid * NB, NB)], idx_v)
    pltpu.sync_copy(ones_hbm, ones_v)
    pltpu.sync_copy(zeros_hbm.at[pl.ds(sid * 640, 640)],
                    acc.at[pl.ds(sid * 640, 640)])
    plsc.subcore_barrier()

    def body(j, carry):
        pltpu.sync_copy(ones_v, acc.at[idx_v.at[j]], add=True)
        return carry

    lax.fori_loop(0, NB, body, 0)
    plsc.subcore_barrier()
    pltpu.sync_copy(acc.at[pl.ds(sid * 640, 640)],
                    out_hbm.at[cid, pl.ds(sid * 640, 640)])


def _make_scatter(D):
    @functools.partial(
        pl.kernel,
        out_type=jax.ShapeDtypeStruct((NC, NPAD, D), jnp.float32),
        mesh=_MESH,
        scratch_types=[
            pltpu.VMEM((NB, B), jnp.int32),
            pltpu.VMEM((NB, B), jnp.int32),
            pltpu.VMEM((B, D), jnp.float32),
            pltpu.VMEM((B, D), jnp.float32),
            pltpu.VMEM_SHARED((NPAD, D), jnp.float32),
            pltpu.SemaphoreType.DMA,
            pltpu.SemaphoreType.DMA,
        ],
        compiler_params=pltpu.CompilerParams(use_tc_tiling_on_sc=False),
    )
    def _scatter(g_hbm, src2d_hbm, dst2d_hbm, zeros_hbm, out_hbm,
                 srcv, dstv, buf0, buf1, acc, sem0, sem1):
        cid = lax.axis_index("c")
        sid = lax.axis_index("s")
        tid = cid * NS + sid
        pltpu.sync_copy(src2d_hbm.at[pl.ds(tid * NB, NB)], srcv)
        pltpu.sync_copy(dst2d_hbm.at[pl.ds(tid * NB, NB)], dstv)
        pltpu.sync_copy(zeros_hbm.at[pl.ds(sid * ROWS_PT, ROWS_PT)],
                        acc.at[pl.ds(sid * ROWS_PT, ROWS_PT)])
        plsc.subcore_barrier()

        bufs = (buf0, buf1)
        sems = (sem0, sem1)
        pltpu.async_copy(g_hbm.at[srcv.at[0]], buf0, sem0)

        def body(j, carry):
            @pl.when(j % 2 == 0)
            def _():
                @pl.when(j + 1 < NB)
                def _():
                    pltpu.async_copy(g_hbm.at[srcv.at[j + 1]], buf1, sem1)

                pltpu.make_async_copy(g_hbm.at[srcv.at[j]], buf0, sem0).wait()
                pltpu.sync_copy(buf0, acc.at[dstv.at[j]], add=True)

            @pl.when(j % 2 == 1)
            def _():
                @pl.when(j + 1 < NB)
                def _():
                    pltpu.async_copy(g_hbm.at[srcv.at[j + 1]], buf0, sem0)

                pltpu.make_async_copy(g_hbm.at[srcv.at[j]], buf1, sem1).wait()
                pltpu.sync_copy(buf1, acc.at[dstv.at[j]], add=True)

            return carry

        lax.fori_loop(0, NB, body, 0)
        plsc.subcore_barrier()
        pltpu.sync_copy(acc.at[pl.ds(sid * ROWS_PT, ROWS_PT)],
                        out_hbm.at[cid, pl.ds(sid * ROWS_PT, ROWS_PT)])

    return _scatter


HH = HID // 2
_scatter_hid = _make_scatter(HH)
_scatter_out = _make_scatter(C)


_R = 2000


def _tc1_body(x_ref, w_ref, degt_ref, g1a_ref, g1b_ref, dinv_ref):
    d2 = degt_ref[...]
    deg = d2[:, 0:1] + d2[:, 1:2] + 1.0
    dinv = lax.rsqrt(deg)
    h = jnp.dot(x_ref[...], w_ref[...], preferred_element_type=jnp.float32)
    g1 = h * dinv
    g1a_ref[...] = g1[:, :HH]
    g1b_ref[...] = g1[:, HH:]
    dinv_ref[...] = dinv


def _tc1(x, w1, degt):
    return pl.pallas_call(
        _tc1_body,
        grid=(N // _R,),
        in_specs=[
            pl.BlockSpec((_R, F_IN), lambda i: (i, 0)),
            pl.BlockSpec((F_IN, HID), lambda i: (0, 0)),
            pl.BlockSpec((_R, NC), lambda i: (i, 0)),
        ],
        out_specs=[
            pl.BlockSpec((_R, HH), lambda i: (i, 0)),
            pl.BlockSpec((_R, HH), lambda i: (i, 0)),
            pl.BlockSpec((_R, 1), lambda i: (i, 0)),
        ],
        out_shape=[
            jax.ShapeDtypeStruct((N, HH), jnp.float32),
            jax.ShapeDtypeStruct((N, HH), jnp.float32),
            jax.ShapeDtypeStruct((N, 1), jnp.float32),
        ],
    )(x, w1, degt)


def _tc2_body(acca_ref, accb_ref, g1a_ref, g1b_ref, dinv_ref, b1_ref, w2_ref,
              g2_ref):
    a = jnp.concatenate(
        [acca_ref[0] + acca_ref[1] + g1a_ref[...],
         accb_ref[0] + accb_ref[1] + g1b_ref[...]], axis=1)
    y = a * dinv_ref[...] + b1_ref[...]
    h = jnp.maximum(y, 0.0)
    g2_ref[...] = jnp.dot(h, w2_ref[...],
                          preferred_element_type=jnp.float32) * dinv_ref[...]


def _tc2(acc1a, acc1b, g1a, g1b, dinv, b1, w2):
    return pl.pallas_call(
        _tc2_body,
        grid=(N // _R,),
        in_specs=[
            pl.BlockSpec((NC, _R, HH), lambda i: (0, i, 0)),
            pl.BlockSpec((NC, _R, HH), lambda i: (0, i, 0)),
            pl.BlockSpec((_R, HH), lambda i: (i, 0)),
            pl.BlockSpec((_R, HH), lambda i: (i, 0)),
            pl.BlockSpec((_R, 1), lambda i: (i, 0)),
            pl.BlockSpec((1, HID), lambda i: (0, 0)),
            pl.BlockSpec((HID, C), lambda i: (0, 0)),
        ],
        out_specs=pl.BlockSpec((_R, C), lambda i: (i, 0)),
        out_shape=jax.ShapeDtypeStruct((N, C), jnp.float32),
    )(acc1a, acc1b, g1a, g1b, dinv, b1, w2)


def _tc3_body(acc_ref, g2_ref, dinv_ref, b2_ref, o_ref):
    a = acc_ref[0] + acc_ref[1] + g2_ref[...]
    y = a * dinv_ref[...] + b2_ref[...]
    m = jnp.max(y, axis=1, keepdims=True)
    e = jnp.exp(y - m)
    s = jnp.sum(e, axis=1, keepdims=True)
    o_ref[...] = y - m - jnp.log(s)


def _tc3(acc2, g2, dinv, b2):
    return pl.pallas_call(
        _tc3_body,
        grid=(N // _R,),
        in_specs=[
            pl.BlockSpec((NC, _R, C), lambda i: (0, i, 0)),
            pl.BlockSpec((_R, C), lambda i: (i, 0)),
            pl.BlockSpec((_R, 1), lambda i: (i, 0)),
            pl.BlockSpec((1, C), lambda i: (0, 0)),
        ],
        out_specs=pl.BlockSpec((_R, C), lambda i: (i, 0)),
        out_shape=jax.ShapeDtypeStruct((N, C), jnp.float32),
    )(acc2, g2, dinv, b2)


def kernel(x, edge_index, W1, b1, W2, b2):
    src2d = edge_index[0].reshape(E // B, B)
    dst2d = edge_index[1].reshape(E // B, B)
    ones_b = jnp.ones((B,), jnp.float32)
    zeros_n = jnp.zeros((NPAD,), jnp.float32)
    zeros_h = jnp.zeros((NPAD, HH), jnp.float32)
    zeros_c = jnp.zeros((NPAD, C), jnp.float32)

    degp = _deg_kernel(dst2d, ones_b, zeros_n)
    g1a, g1b, dinv = _tc1(x, W1, degp[:, :N].T)
    acc1a = _scatter_hid(g1a, src2d, dst2d, zeros_h)
    acc1b = _scatter_hid(g1b, src2d, dst2d, zeros_h)
    g2 = _tc2(acc1a, acc1b, g1a, g1b, dinv,
              b1.reshape(1, HID), W2)
    acc2 = _scatter_out(g2, src2d, dst2d, zeros_c)
    return _tc3(acc2, g2, dinv, b2.reshape(1, C))


# speedup vs baseline: 30.6322x; 1.0015x over previous
import functools

import jax
import jax.numpy as jnp
from jax import lax
from jax.experimental import pallas as pl
from jax.experimental.pallas import tpu as pltpu
from jax.experimental.pallas import tpu_sc as plsc

N = 10000
E = 320000
F_IN = 128
HID = 128
C = 40

NC = 2
NS = 16
NT = NC * NS
B = 125
EPT = E // NT
NB = EPT // B
NPAD = 10240
ROWS_PT = NPAD // NS

_MESH = plsc.VectorSubcoreMesh(core_axis_name="c", subcore_axis_name="s")


@functools.partial(
    pl.kernel,
    out_type=jax.ShapeDtypeStruct((NC, NPAD), jnp.float32),
    mesh=_MESH,
    scratch_types=[
        pltpu.VMEM((NB, B), jnp.int32),
        pltpu.VMEM((B,), jnp.float32),
        pltpu.VMEM_SHARED((NPAD,), jnp.float32),
    ],
)
def _deg_kernel(dst2d_hbm, ones_hbm, zeros_hbm, out_hbm, idx_v, ones_v, acc):
    cid = lax.axis_index("c")
    sid = lax.axis_index("s")
    tid = cid * NS + sid
    pltpu.sync_copy(dst2d_hbm.at[pl.ds(tid * NB, NB)], idx_v)
    pltpu.sync_copy(ones_hbm, ones_v)
    pltpu.sync_copy(zeros_hbm.at[pl.ds(sid * 640, 640)],
                    acc.at[pl.ds(sid * 640, 640)])
    plsc.subcore_barrier()

    def body(j, carry):
        pltpu.sync_copy(ones_v, acc.at[idx_v.at[j]], add=True)
        return carry

    lax.fori_loop(0, NB, body, 0)
    plsc.subcore_barrier()
    pltpu.sync_copy(acc.at[pl.ds(sid * 640, 640)],
                    out_hbm.at[cid, pl.ds(sid * 640, 640)])


def _make_scatter(D):
    @functools.partial(
        pl.kernel,
        out_type=jax.ShapeDtypeStruct((NC, NPAD, D), jnp.float32),
        mesh=_MESH,
        scratch_types=[
            pltpu.VMEM((NB, B), jnp.int32),
            pltpu.VMEM((NB, B), jnp.int32),
            pltpu.VMEM((B, D), jnp.float32),
            pltpu.VMEM((B, D), jnp.float32),
            pltpu.VMEM_SHARED((NPAD, D), jnp.float32),
            pltpu.SemaphoreType.DMA,
            pltpu.SemaphoreType.DMA,
            pltpu.SemaphoreType.DMA,
            pltpu.SemaphoreType.DMA,
        ],
        compiler_params=pltpu.CompilerParams(use_tc_tiling_on_sc=False),
    )
    def _scatter(g_hbm, src2d_hbm, dst2d_hbm, zeros_hbm, out_hbm,
                 srcv, dstv, buf0, buf1, acc, gsem0, gsem1, ssem0, ssem1):
        cid = lax.axis_index("c")
        sid = lax.axis_index("s")
        tid = cid * NS + sid
        pltpu.sync_copy(src2d_hbm.at[pl.ds(tid * NB, NB)], srcv)
        pltpu.sync_copy(dst2d_hbm.at[pl.ds(tid * NB, NB)], dstv)
        pltpu.sync_copy(zeros_hbm.at[pl.ds(sid * ROWS_PT, ROWS_PT)],
                        acc.at[pl.ds(sid * ROWS_PT, ROWS_PT)])
        plsc.subcore_barrier()

        bufs = (buf0, buf1)
        gsems = (gsem0, gsem1)
        ssems = (ssem0, ssem1)
        pltpu.async_copy(g_hbm.at[srcv.at[0]], buf0, gsem0)

        def half(j, cur, nxt):
            @pl.when(j >= 1)
            def _():
                pltpu.make_async_copy(bufs[nxt], acc.at[dstv.at[j]],
                                      ssems[nxt]).wait()

            @pl.when(j + 1 < NB)
            def _():
                pltpu.async_copy(g_hbm.at[srcv.at[j + 1]], bufs[nxt], gsems[nxt])

            pltpu.make_async_copy(g_hbm.at[srcv.at[j]], bufs[cur], gsems[cur]).wait()
            pltpu.async_copy(bufs[cur], acc.at[dstv.at[j]], ssems[cur], add=True)

        def body(j, carry):
            @pl.when(j % 2 == 0)
            def _():
                half(j, 0, 1)

            @pl.when(j % 2 == 1)
            def _():
                half(j, 1, 0)

            return carry

        lax.fori_loop(0, NB, body, 0)
        pltpu.make_async_copy(buf1, acc.at[dstv.at[0]], ssem1).wait()
        plsc.subcore_barrier()
        pltpu.sync_copy(acc.at[pl.ds(sid * ROWS_PT, ROWS_PT)],
                        out_hbm.at[cid, pl.ds(sid * ROWS_PT, ROWS_PT)])

    return _scatter


HH = HID // 2
_scatter_hid = _make_scatter(HH)
_scatter_out = _make_scatter(C)


_R = 2000


def _tc1_body(x_ref, w_ref, degt_ref, g1a_ref, g1b_ref, dinv_ref):
    d2 = degt_ref[...]
    deg = d2[:, 0:1] + d2[:, 1:2] + 1.0
    dinv = lax.rsqrt(deg)
    h = jnp.dot(x_ref[...], w_ref[...], preferred_element_type=jnp.float32)
    g1 = h * dinv
    g1a_ref[...] = g1[:, :HH]
    g1b_ref[...] = g1[:, HH:]
    dinv_ref[...] = dinv


def _tc1(x, w1, degt):
    return pl.pallas_call(
        _tc1_body,
        grid=(N // _R,),
        in_specs=[
            pl.BlockSpec((_R, F_IN), lambda i: (i, 0)),
            pl.BlockSpec((F_IN, HID), lambda i: (0, 0)),
            pl.BlockSpec((_R, NC), lambda i: (i, 0)),
        ],
        out_specs=[
            pl.BlockSpec((_R, HH), lambda i: (i, 0)),
            pl.BlockSpec((_R, HH), lambda i: (i, 0)),
            pl.BlockSpec((_R, 1), lambda i: (i, 0)),
        ],
        out_shape=[
            jax.ShapeDtypeStruct((N, HH), jnp.float32),
            jax.ShapeDtypeStruct((N, HH), jnp.float32),
            jax.ShapeDtypeStruct((N, 1), jnp.float32),
        ],
    )(x, w1, degt)


def _tc2_body(acca_ref, accb_ref, g1a_ref, g1b_ref, dinv_ref, b1_ref, w2_ref,
              g2_ref):
    a = jnp.concatenate(
        [acca_ref[0] + acca_ref[1] + g1a_ref[...],
         accb_ref[0] + accb_ref[1] + g1b_ref[...]], axis=1)
    y = a * dinv_ref[...] + b1_ref[...]
    h = jnp.maximum(y, 0.0)
    g2_ref[...] = jnp.dot(h, w2_ref[...],
                          preferred_element_type=jnp.float32) * dinv_ref[...]


def _tc2(acc1a, acc1b, g1a, g1b, dinv, b1, w2):
    return pl.pallas_call(
        _tc2_body,
        grid=(N // _R,),
        in_specs=[
            pl.BlockSpec((NC, _R, HH), lambda i: (0, i, 0)),
            pl.BlockSpec((NC, _R, HH), lambda i: (0, i, 0)),
            pl.BlockSpec((_R, HH), lambda i: (i, 0)),
            pl.BlockSpec((_R, HH), lambda i: (i, 0)),
            pl.BlockSpec((_R, 1), lambda i: (i, 0)),
            pl.BlockSpec((1, HID), lambda i: (0, 0)),
            pl.BlockSpec((HID, C), lambda i: (0, 0)),
        ],
        out_specs=pl.BlockSpec((_R, C), lambda i: (i, 0)),
        out_shape=jax.ShapeDtypeStruct((N, C), jnp.float32),
    )(acc1a, acc1b, g1a, g1b, dinv, b1, w2)


def _tc3_body(acc_ref, g2_ref, dinv_ref, b2_ref, o_ref):
    a = acc_ref[0] + acc_ref[1] + g2_ref[...]
    y = a * dinv_ref[...] + b2_ref[...]
    m = jnp.max(y, axis=1, keepdims=True)
    e = jnp.exp(y - m)
    s = jnp.sum(e, axis=1, keepdims=True)
    o_ref[...] = y - m - jnp.log(s)


def _tc3(acc2, g2, dinv, b2):
    return pl.pallas_call(
        _tc3_body,
        grid=(N // _R,),
        in_specs=[
            pl.BlockSpec((NC, _R, C), lambda i: (0, i, 0)),
            pl.BlockSpec((_R, C), lambda i: (i, 0)),
            pl.BlockSpec((_R, 1), lambda i: (i, 0)),
            pl.BlockSpec((1, C), lambda i: (0, 0)),
        ],
        out_specs=pl.BlockSpec((_R, C), lambda i: (i, 0)),
        out_shape=jax.ShapeDtypeStruct((N, C), jnp.float32),
    )(acc2, g2, dinv, b2)


def kernel(x, edge_index, W1, b1, W2, b2):
    src2d = edge_index[0].reshape(E // B, B)
    dst2d = edge_index[1].reshape(E // B, B)
    ones_b = jnp.ones((B,), jnp.float32)
    zeros_n = jnp.zeros((NPAD,), jnp.float32)
    zeros_h = jnp.zeros((NPAD, HH), jnp.float32)
    zeros_c = jnp.zeros((NPAD, C), jnp.float32)

    degp = _deg_kernel(dst2d, ones_b, zeros_n)
    g1a, g1b, dinv = _tc1(x, W1, degp[:, :N].T)
    acc1a = _scatter_hid(g1a, src2d, dst2d, zeros_h)
    acc1b = _scatter_hid(g1b, src2d, dst2d, zeros_h)
    g2 = _tc2(acc1a, acc1b, g1a, g1b, dinv,
              b1.reshape(1, HID), W2)
    acc2 = _scatter_out(g2, src2d, dst2d, zeros_c)
    return _tc3(acc2, g2, dinv, b2.reshape(1, C))


# speedup vs baseline: 30.6491x; 1.0006x over previous
import functools

import jax
import jax.numpy as jnp
from jax import lax
from jax.experimental import pallas as pl
from jax.experimental.pallas import tpu as pltpu
from jax.experimental.pallas import tpu_sc as plsc

N = 10000
E = 320000
F_IN = 128
HID = 128
C = 40

NC = 2
NS = 16
NT = NC * NS
B = 125
EPT = E // NT
NB = EPT // B
NPAD = 10240
ROWS_PT = NPAD // NS

_MESH = plsc.VectorSubcoreMesh(core_axis_name="c", subcore_axis_name="s")


@functools.partial(
    pl.kernel,
    out_type=jax.ShapeDtypeStruct((NC, NPAD), jnp.float32),
    mesh=_MESH,
    scratch_types=[
        pltpu.VMEM((NB, B), jnp.int32),
        pltpu.VMEM((B,), jnp.float32),
        pltpu.VMEM_SHARED((NPAD,), jnp.float32),
    ],
)
def _deg_kernel(dst2d_hbm, ones_hbm, zeros_hbm, out_hbm, idx_v, ones_v, acc):
    cid = lax.axis_index("c")
    sid = lax.axis_index("s")
    tid = cid * NS + sid
    pltpu.sync_copy(dst2d_hbm.at[pl.ds(tid * NB, NB)], idx_v)
    pltpu.sync_copy(ones_hbm, ones_v)
    pltpu.sync_copy(zeros_hbm.at[pl.ds(sid * 640, 640)],
                    acc.at[pl.ds(sid * 640, 640)])
    plsc.subcore_barrier()

    def body(j, carry):
        pltpu.sync_copy(ones_v, acc.at[idx_v.at[j]], add=True)
        return carry

    lax.fori_loop(0, NB, body, 0)
    plsc.subcore_barrier()
    pltpu.sync_copy(acc.at[pl.ds(sid * 640, 640)],
                    out_hbm.at[cid, pl.ds(sid * 640, 640)])


def _scatter_phase(g_hbm, out_hbm, zeros_hbm, srcv, dstv, bufs, gsems, ssems,
                   acc, cid, sid):
    pltpu.sync_copy(zeros_hbm.at[pl.ds(sid * ROWS_PT, ROWS_PT)],
                    acc.at[pl.ds(sid * ROWS_PT, ROWS_PT)])
    plsc.subcore_barrier()
    pltpu.async_copy(g_hbm.at[srcv.at[0]], bufs[0], gsems[0])

    def half(j, cur, nxt):
        @pl.when(j >= 1)
        def _():
            pltpu.make_async_copy(bufs[nxt], acc.at[dstv.at[j]],
                                  ssems[nxt]).wait()

        @pl.when(j + 1 < NB)
        def _():
            pltpu.async_copy(g_hbm.at[srcv.at[j + 1]], bufs[nxt], gsems[nxt])

        pltpu.make_async_copy(g_hbm.at[srcv.at[j]], bufs[cur], gsems[cur]).wait()
        pltpu.async_copy(bufs[cur], acc.at[dstv.at[j]], ssems[cur], add=True)

    def body(j, carry):
        @pl.when(j % 2 == 0)
        def _():
            half(j, 0, 1)

        @pl.when(j % 2 == 1)
        def _():
            half(j, 1, 0)

        return carry

    lax.fori_loop(0, NB, body, 0)
    pltpu.make_async_copy(bufs[1], acc.at[dstv.at[0]], ssems[1]).wait()
    plsc.subcore_barrier()
    pltpu.sync_copy(acc.at[pl.ds(sid * ROWS_PT, ROWS_PT)],
                    out_hbm.at[cid, pl.ds(sid * ROWS_PT, ROWS_PT)])


def _make_scatter(D, num_tables):
    @functools.partial(
        pl.kernel,
        out_type=[jax.ShapeDtypeStruct((NC, NPAD, D), jnp.float32)
                  for _ in range(num_tables)],
        mesh=_MESH,
        scratch_types=[
            pltpu.VMEM((NB, B), jnp.int32),
            pltpu.VMEM((NB, B), jnp.int32),
            pltpu.VMEM((B, D), jnp.float32),
            pltpu.VMEM((B, D), jnp.float32),
            pltpu.VMEM_SHARED((NPAD, D), jnp.float32),
            pltpu.SemaphoreType.DMA,
            pltpu.SemaphoreType.DMA,
            pltpu.SemaphoreType.DMA,
            pltpu.SemaphoreType.DMA,
        ],
        compiler_params=pltpu.CompilerParams(use_tc_tiling_on_sc=False),
    )
    def _scatter(*args):
        tables = args[:num_tables]
        src2d_hbm, dst2d_hbm, zeros_hbm = args[num_tables:num_tables + 3]
        outs = args[num_tables + 3:2 * num_tables + 3]
        (srcv, dstv, buf0, buf1, acc,
         gsem0, gsem1, ssem0, ssem1) = args[2 * num_tables + 3:]
        cid = lax.axis_index("c")
        sid = lax.axis_index("s")
        tid = cid * NS + sid
        pltpu.sync_copy(src2d_hbm.at[pl.ds(tid * NB, NB)], srcv)
        pltpu.sync_copy(dst2d_hbm.at[pl.ds(tid * NB, NB)], dstv)
        for g_hbm, out_hbm in zip(tables, outs):
            _scatter_phase(g_hbm, out_hbm, zeros_hbm, srcv, dstv,
                           (buf0, buf1), (gsem0, gsem1), (ssem0, ssem1),
                           acc, cid, sid)

    return _scatter


HH = HID // 2
_scatter_hid = _make_scatter(HH, 2)
_scatter_out = _make_scatter(C, 1)


_R = 2000


def _tc1a_body(x_ref, w_ref, h_ref):
    h_ref[...] = jnp.dot(x_ref[...], w_ref[...],
                         preferred_element_type=jnp.float32)


def _tc1a(x, w1):
    return pl.pallas_call(
        _tc1a_body,
        grid=(N // _R,),
        in_specs=[
            pl.BlockSpec((_R, F_IN), lambda i: (i, 0)),
            pl.BlockSpec((F_IN, HID), lambda i: (0, 0)),
        ],
        out_specs=pl.BlockSpec((_R, HID), lambda i: (i, 0)),
        out_shape=jax.ShapeDtypeStruct((N, HID), jnp.float32),
    )(x, w1)


def _tc1b_body(h_ref, degt_ref, g1a_ref, g1b_ref, dinv_ref):
    d2 = degt_ref[...]
    deg = d2[:, 0:1] + d2[:, 1:2] + 1.0
    dinv = lax.rsqrt(deg)
    g1 = h_ref[...] * dinv
    g1a_ref[...] = g1[:, :HH]
    g1b_ref[...] = g1[:, HH:]
    dinv_ref[...] = dinv


def _tc1b(h, degt):
    return pl.pallas_call(
        _tc1b_body,
        grid=(N // _R,),
        in_specs=[
            pl.BlockSpec((_R, HID), lambda i: (i, 0)),
            pl.BlockSpec((_R, NC), lambda i: (i, 0)),
        ],
        out_specs=[
            pl.BlockSpec((_R, HH), lambda i: (i, 0)),
            pl.BlockSpec((_R, HH), lambda i: (i, 0)),
            pl.BlockSpec((_R, 1), lambda i: (i, 0)),
        ],
        out_shape=[
            jax.ShapeDtypeStruct((N, HH), jnp.float32),
            jax.ShapeDtypeStruct((N, HH), jnp.float32),
            jax.ShapeDtypeStruct((N, 1), jnp.float32),
        ],
    )(h, degt)


def _tc2_body(acca_ref, accb_ref, g1a_ref, g1b_ref, dinv_ref, b1_ref, w2_ref,
              g2_ref):
    a = jnp.concatenate(
        [acca_ref[0] + acca_ref[1] + g1a_ref[...],
         accb_ref[0] + accb_ref[1] + g1b_ref[...]], axis=1)
    y = a * dinv_ref[...] + b1_ref[...]
    h = jnp.maximum(y, 0.0)
    g2_ref[...] = jnp.dot(h, w2_ref[...],
                          preferred_element_type=jnp.float32) * dinv_ref[...]


def _tc2(acc1a, acc1b, g1a, g1b, dinv, b1, w2):
    return pl.pallas_call(
        _tc2_body,
        grid=(N // _R,),
        in_specs=[
            pl.BlockSpec((NC, _R, HH), lambda i: (0, i, 0)),
            pl.BlockSpec((NC, _R, HH), lambda i: (0, i, 0)),
            pl.BlockSpec((_R, HH), lambda i: (i, 0)),
            pl.BlockSpec((_R, HH), lambda i: (i, 0)),
            pl.BlockSpec((_R, 1), lambda i: (i, 0)),
            pl.BlockSpec((1, HID), lambda i: (0, 0)),
            pl.BlockSpec((HID, C), lambda i: (0, 0)),
        ],
        out_specs=pl.BlockSpec((_R, C), lambda i: (i, 0)),
        out_shape=jax.ShapeDtypeStruct((N, C), jnp.float32),
    )(acc1a, acc1b, g1a, g1b, dinv, b1, w2)


def _tc3_body(acc_ref, g2_ref, dinv_ref, b2_ref, o_ref):
    a = acc_ref[0] + acc_ref[1] + g2_ref[...]
    y = a * dinv_ref[...] + b2_ref[...]
    m = jnp.max(y, axis=1, keepdims=True)
    e = jnp.exp(y - m)
    s = jnp.sum(e, axis=1, keepdims=True)
    o_ref[...] = y - m - jnp.log(s)


def _tc3(acc2, g2, dinv, b2):
    return pl.pallas_call(
        _tc3_body,
        grid=(N // _R,),
        in_specs=[
            pl.BlockSpec((NC, _R, C), lambda i: (0, i, 0)),
            pl.BlockSpec((_R, C), lambda i: (i, 0)),
            pl.BlockSpec((_R, 1), lambda i: (i, 0)),
            pl.BlockSpec((1, C), lambda i: (0, 0)),
        ],
        out_specs=pl.BlockSpec((_R, C), lambda i: (i, 0)),
        out_shape=jax.ShapeDtypeStruct((N, C), jnp.float32),
    )(acc2, g2, dinv, b2)


def kernel(x, edge_index, W1, b1, W2, b2):
    src2d = edge_index[0].reshape(E // B, B)
    dst2d = edge_index[1].reshape(E // B, B)
    ones_b = jnp.ones((B,), jnp.float32)
    zeros_n = jnp.zeros((NPAD,), jnp.float32)
    zeros_h = jnp.zeros((NPAD, HH), jnp.float32)
    zeros_c = jnp.zeros((NPAD, C), jnp.float32)

    degp = _deg_kernel(dst2d, ones_b, zeros_n)
    h1 = _tc1a(x, W1)
    g1a, g1b, dinv = _tc1b(h1, degp[:, :N].T)
    acc1a, acc1b = _scatter_hid(g1a, g1b, src2d, dst2d, zeros_h)
    g2 = _tc2(acc1a, acc1b, g1a, g1b, dinv,
              b1.reshape(1, HID), W2)
    (acc2,) = _scatter_out(g2, src2d, dst2d, zeros_c)
    return _tc3(acc2, g2, dinv, b2.reshape(1, C))


# speedup vs baseline: 34.8685x; 1.1377x over previous
import functools

import jax
import jax.numpy as jnp
from jax import lax
from jax.experimental import pallas as pl
from jax.experimental.pallas import tpu as pltpu
from jax.experimental.pallas import tpu_sc as plsc

N = 10000
E = 320000
F_IN = 128
HID = 128
C = 40

NC = 2
NS = 16
NT = NC * NS
B = 125
EPT = E // NT
NB = EPT // B
NPAD = 10240
ROWS_PT = NPAD // NS

_MESH = plsc.VectorSubcoreMesh(core_axis_name="c", subcore_axis_name="s")


@functools.partial(
    pl.kernel,
    out_type=jax.ShapeDtypeStruct((NC, NPAD), jnp.float32),
    mesh=_MESH,
    scratch_types=[
        pltpu.VMEM((NB, B), jnp.int32),
        pltpu.VMEM((B,), jnp.float32),
        pltpu.VMEM_SHARED((NPAD,), jnp.float32),
    ],
)
def _deg_kernel(dst2d_hbm, ones_hbm, zeros_hbm, out_hbm, idx_v, ones_v, acc):
    cid = lax.axis_index("c")
    sid = lax.axis_index("s")
    tid = cid * NS + sid
    pltpu.sync_copy(dst2d_hbm.at[pl.ds(tid * NB, NB)], idx_v)
    pltpu.sync_copy(ones_hbm, ones_v)
    pltpu.sync_copy(zeros_hbm.at[pl.ds(sid * 640, 640)],
                    acc.at[pl.ds(sid * 640, 640)])
    plsc.subcore_barrier()

    def body(j, carry):
        pltpu.sync_copy(ones_v, acc.at[idx_v.at[j]], add=True)
        return carry

    lax.fori_loop(0, NB, body, 0)
    plsc.subcore_barrier()
    pltpu.sync_copy(acc.at[pl.ds(sid * 640, 640)],
                    out_hbm.at[cid, pl.ds(sid * 640, 640)])


def _scatter_phase(g_hbm, out_hbm, zeros_hbm, srcv, dstv, bufs, gsems, ssems,
                   acc, cid, sid):
    pltpu.sync_copy(zeros_hbm.at[pl.ds(sid * ROWS_PT, ROWS_PT)],
                    acc.at[pl.ds(sid * ROWS_PT, ROWS_PT)])
    plsc.subcore_barrier()
    pltpu.async_copy(g_hbm.at[srcv.at[0]], bufs[0], gsems[0])
    pltpu.async_copy(g_hbm.at[srcv.at[1]], bufs[1], gsems[1])

    def body(jj, carry):
        for k in range(4):
            j = 4 * jj + k
            nk = (k + 2) % 4

            @pl.when(j >= 2)
            def _():
                pltpu.make_async_copy(bufs[nk], acc.at[dstv.at[j]],
                                      ssems[nk]).wait()

            @pl.when(j + 2 < NB)
            def _():
                pltpu.async_copy(g_hbm.at[srcv.at[j + 2]], bufs[nk], gsems[nk])

            pltpu.make_async_copy(g_hbm.at[srcv.at[j]], bufs[k], gsems[k]).wait()
            pltpu.async_copy(bufs[k], acc.at[dstv.at[j]], ssems[k], add=True)
        return carry

    lax.fori_loop(0, NB // 4, body, 0)
    pltpu.make_async_copy(bufs[2], acc.at[dstv.at[0]], ssems[2]).wait()
    pltpu.make_async_copy(bufs[3], acc.at[dstv.at[0]], ssems[3]).wait()
    plsc.subcore_barrier()
    pltpu.sync_copy(acc.at[pl.ds(sid * ROWS_PT, ROWS_PT)],
                    out_hbm.at[cid, pl.ds(sid * ROWS_PT, ROWS_PT)])


def _make_scatter(D, num_tables):
    @functools.partial(
        pl.kernel,
        out_type=[jax.ShapeDtypeStruct((NC, NPAD, D), jnp.float32)
                  for _ in range(num_tables)],
        mesh=_MESH,
        scratch_types=[
            pltpu.VMEM((NB, B), jnp.int32),
            pltpu.VMEM((NB, B), jnp.int32),
            pltpu.VMEM((B, D), jnp.float32),
            pltpu.VMEM((B, D), jnp.float32),
            pltpu.VMEM((B, D), jnp.float32),
            pltpu.VMEM((B, D), jnp.float32),
            pltpu.VMEM_SHARED((NPAD, D), jnp.float32),
        ] + [pltpu.SemaphoreType.DMA] * 8,
        compiler_params=pltpu.CompilerParams(use_tc_tiling_on_sc=False),
    )
    def _scatter(*args):
        tables = args[:num_tables]
        src2d_hbm, dst2d_hbm, zeros_hbm = args[num_tables:num_tables + 3]
        outs = args[num_tables + 3:2 * num_tables + 3]
        scratch = args[2 * num_tables + 3:]
        srcv, dstv = scratch[0], scratch[1]
        bufs = scratch[2:6]
        acc = scratch[6]
        gsems = scratch[7:11]
        ssems = scratch[11:15]
        cid = lax.axis_index("c")
        sid = lax.axis_index("s")
        tid = cid * NS + sid
        pltpu.sync_copy(src2d_hbm.at[pl.ds(tid * NB, NB)], srcv)
        pltpu.sync_copy(dst2d_hbm.at[pl.ds(tid * NB, NB)], dstv)
        for g_hbm, out_hbm in zip(tables, outs):
            _scatter_phase(g_hbm, out_hbm, zeros_hbm, srcv, dstv,
                           bufs, gsems, ssems, acc, cid, sid)

    return _scatter


HH = HID // 2
_scatter_hid = _make_scatter(HH, 2)
_scatter_out = _make_scatter(C, 1)


_R = 2000


def _tc1a_body(x_ref, w_ref, h_ref):
    h_ref[...] = jnp.dot(x_ref[...], w_ref[...],
                         preferred_element_type=jnp.float32)


def _tc1a(x, w1):
    return pl.pallas_call(
        _tc1a_body,
        grid=(N // _R,),
        in_specs=[
            pl.BlockSpec((_R, F_IN), lambda i: (i, 0)),
            pl.BlockSpec((F_IN, HID), lambda i: (0, 0)),
        ],
        out_specs=pl.BlockSpec((_R, HID), lambda i: (i, 0)),
        out_shape=jax.ShapeDtypeStruct((N, HID), jnp.float32),
    )(x, w1)


def _tc1b_body(h_ref, degt_ref, g1a_ref, g1b_ref, dinv_ref):
    d2 = degt_ref[...]
    deg = d2[:, 0:1] + d2[:, 1:2] + 1.0
    dinv = lax.rsqrt(deg)
    g1 = h_ref[...] * dinv
    g1a_ref[...] = g1[:, :HH]
    g1b_ref[...] = g1[:, HH:]
    dinv_ref[...] = dinv


def _tc1b(h, degt):
    return pl.pallas_call(
        _tc1b_body,
        grid=(N // _R,),
        in_specs=[
            pl.BlockSpec((_R, HID), lambda i: (i, 0)),
            pl.BlockSpec((_R, NC), lambda i: (i, 0)),
        ],
        out_specs=[
            pl.BlockSpec((_R, HH), lambda i: (i, 0)),
            pl.BlockSpec((_R, HH), lambda i: (i, 0)),
            pl.BlockSpec((_R, 1), lambda i: (i, 0)),
        ],
        out_shape=[
            jax.ShapeDtypeStruct((N, HH), jnp.float32),
            jax.ShapeDtypeStruct((N, HH), jnp.float32),
            jax.ShapeDtypeStruct((N, 1), jnp.float32),
        ],
    )(h, degt)


def _tc2_body(acca_ref, accb_ref, g1a_ref, g1b_ref, dinv_ref, b1_ref, w2_ref,
              g2_ref):
    a = jnp.concatenate(
        [acca_ref[0] + acca_ref[1] + g1a_ref[...],
         accb_ref[0] + accb_ref[1] + g1b_ref[...]], axis=1)
    y = a * dinv_ref[...] + b1_ref[...]
    h = jnp.maximum(y, 0.0)
    g2_ref[...] = jnp.dot(h, w2_ref[...],
                          preferred_element_type=jnp.float32) * dinv_ref[...]


def _tc2(acc1a, acc1b, g1a, g1b, dinv, b1, w2):
    return pl.pallas_call(
        _tc2_body,
        grid=(N // _R,),
        in_specs=[
            pl.BlockSpec((NC, _R, HH), lambda i: (0, i, 0)),
            pl.BlockSpec((NC, _R, HH), lambda i: (0, i, 0)),
            pl.BlockSpec((_R, HH), lambda i: (i, 0)),
            pl.BlockSpec((_R, HH), lambda i: (i, 0)),
            pl.BlockSpec((_R, 1), lambda i: (i, 0)),
            pl.BlockSpec((1, HID), lambda i: (0, 0)),
            pl.BlockSpec((HID, C), lambda i: (0, 0)),
        ],
        out_specs=pl.BlockSpec((_R, C), lambda i: (i, 0)),
        out_shape=jax.ShapeDtypeStruct((N, C), jnp.float32),
    )(acc1a, acc1b, g1a, g1b, dinv, b1, w2)


def _tc3_body(acc_ref, g2_ref, dinv_ref, b2_ref, o_ref):
    a = acc_ref[0] + acc_ref[1] + g2_ref[...]
    y = a * dinv_ref[...] + b2_ref[...]
    m = jnp.max(y, axis=1, keepdims=True)
    e = jnp.exp(y - m)
    s = jnp.sum(e, axis=1, keepdims=True)
    o_ref[...] = y - m - jnp.log(s)


def _tc3(acc2, g2, dinv, b2):
    return pl.pallas_call(
        _tc3_body,
        grid=(N // _R,),
        in_specs=[
            pl.BlockSpec((NC, _R, C), lambda i: (0, i, 0)),
            pl.BlockSpec((_R, C), lambda i: (i, 0)),
            pl.BlockSpec((_R, 1), lambda i: (i, 0)),
            pl.BlockSpec((1, C), lambda i: (0, 0)),
        ],
        out_specs=pl.BlockSpec((_R, C), lambda i: (i, 0)),
        out_shape=jax.ShapeDtypeStruct((N, C), jnp.float32),
    )(acc2, g2, dinv, b2)


def kernel(x, edge_index, W1, b1, W2, b2):
    src2d = edge_index[0].reshape(E // B, B)
    dst2d = edge_index[1].reshape(E // B, B)
    ones_b = jnp.ones((B,), jnp.float32)
    zeros_n = jnp.zeros((NPAD,), jnp.float32)
    zeros_h = jnp.zeros((NPAD, HH), jnp.float32)
    zeros_c = jnp.zeros((NPAD, C), jnp.float32)

    degp = _deg_kernel(dst2d, ones_b, zeros_n)
    h1 = _tc1a(x, W1)
    g1a, g1b, dinv = _tc1b(h1, degp[:, :N].T)
    acc1a, acc1b = _scatter_hid(g1a, g1b, src2d, dst2d, zeros_h)
    g2 = _tc2(acc1a, acc1b, g1a, g1b, dinv,
              b1.reshape(1, HID), W2)
    (acc2,) = _scatter_out(g2, src2d, dst2d, zeros_c)
    return _tc3(acc2, g2, dinv, b2.reshape(1, C))
